# Initial kernel scaffold; baseline (speedup 1.0000x reference)
import functools

import jax
import jax.numpy as jnp
from jax import lax
from jax.experimental import pallas as pl
from jax.experimental.pallas import tpu as pltpu
from jax.experimental.pallas import tpu_sc as plsc

DIM = 32
IDX_W = 128

_info = plsc.get_sparse_core_info()
NC = _info.num_cores
NS = _info.num_subcores
NW = NC * NS


def _make_lookup(steps):
    mesh = plsc.VectorSubcoreMesh(core_axis_name="c", subcore_axis_name="s")
    n = NW * steps * IDX_W

    @functools.partial(
        pl.kernel,
        mesh=mesh,
        out_type=jax.ShapeDtypeStruct((n, DIM), jnp.float32),
        scratch_types=[
            pltpu.VMEM((steps, IDX_W), jnp.int32),
            pltpu.VMEM((IDX_W, DIM), jnp.float32),
            pltpu.SemaphoreType.DMA,
        ],
    )
    def lookup(table_hbm, idx_hbm, out_hbm, idx_v, rows_v, sem):
        wid = lax.axis_index("s") * NC + lax.axis_index("c")
        base = wid * (steps * IDX_W)
        pltpu.sync_copy(idx_hbm.at[wid], idx_v)

        def step(j, carry):
            pltpu.async_copy(table_hbm.at[idx_v.at[j]], rows_v, sem).wait()
            pltpu.sync_copy(rows_v, out_hbm.at[pl.ds(base + j * IDX_W, IDX_W)])
            return carry

        lax.fori_loop(0, steps, step, 0)

    return lookup


def kernel(table, text_seqs):
    batch, seq = text_seqs.shape
    n = batch * seq
    assert n % (NW * IDX_W) == 0
    steps = n // (NW * IDX_W)
    idx = text_seqs.reshape(NW, steps, IDX_W)
    out = _make_lookup(steps)(table, idx)
    return out.reshape(batch, seq, DIM)


# speedup vs baseline: 2.6971x; 2.6971x over previous
import functools

import jax
import jax.numpy as jnp
from jax import lax
from jax.experimental import pallas as pl
from jax.experimental.pallas import tpu as pltpu
from jax.experimental.pallas import tpu_sc as plsc

DIM = 32
IDX_W = 128

_info = plsc.get_sparse_core_info()
NC = _info.num_cores
NS = _info.num_subcores
NW = NC * NS


def _make_lookup(steps):
    mesh = plsc.VectorSubcoreMesh(core_axis_name="c", subcore_axis_name="s")
    n = NW * steps * IDX_W

    @functools.partial(
        pl.kernel,
        mesh=mesh,
        compiler_params=pltpu.CompilerParams(use_tc_tiling_on_sc=False),
        out_type=jax.ShapeDtypeStruct((n, DIM), jnp.float32),
        scratch_types=[
            pltpu.VMEM((steps, IDX_W), jnp.int32),
            pltpu.VMEM((IDX_W, DIM), jnp.float32),
            pltpu.SemaphoreType.DMA,
        ],
    )
    def lookup(table_hbm, idx_hbm, out_hbm, idx_v, rows_v, sem):
        wid = lax.axis_index("s") * NC + lax.axis_index("c")
        base = wid * (steps * IDX_W)
        pltpu.sync_copy(idx_hbm.at[wid], idx_v)

        def step(j, carry):
            pltpu.async_copy(table_hbm.at[idx_v.at[j]], rows_v, sem).wait()
            pltpu.sync_copy(rows_v, out_hbm.at[pl.ds(base + j * IDX_W, IDX_W)])
            return carry

        lax.fori_loop(0, steps, step, 0)

    return lookup


def kernel(table, text_seqs):
    batch, seq = text_seqs.shape
    n = batch * seq
    assert n % (NW * IDX_W) == 0
    steps = n // (NW * IDX_W)
    idx = text_seqs.reshape(NW, steps, IDX_W)
    out = _make_lookup(steps)(table, idx)
    return out.reshape(batch, seq, DIM)


# speedup vs baseline: 3.0155x; 1.1181x over previous
import functools

import jax
import jax.numpy as jnp
from jax import lax
from jax.experimental import pallas as pl
from jax.experimental.pallas import tpu as pltpu
from jax.experimental.pallas import tpu_sc as plsc

DIM = 32
IDX_W = 128

_info = plsc.get_sparse_core_info()
NC = _info.num_cores
NS = _info.num_subcores
NW = NC * NS


K = 5
NB = 4


def _make_lookup(steps):
    mesh = plsc.VectorSubcoreMesh(core_axis_name="c", subcore_axis_name="s")
    n = NW * steps * IDX_W
    ng = steps // K
    grp = K * IDX_W

    @functools.partial(
        pl.kernel,
        mesh=mesh,
        compiler_params=pltpu.CompilerParams(use_tc_tiling_on_sc=False),
        out_type=jax.ShapeDtypeStruct((n, DIM), jnp.float32),
        scratch_types=[
            pltpu.VMEM((steps, IDX_W), jnp.int32),
            pltpu.VMEM((NB, grp, DIM), jnp.float32),
            pltpu.SemaphoreType.DMA((NB,)),
            pltpu.SemaphoreType.DMA((NB,)),
        ],
    )
    def lookup(table_hbm, idx_hbm, out_hbm, idx_v, rows_v, gsem, ssem):
        wid = lax.axis_index("s") * NC + lax.axis_index("c")
        base = wid * (steps * IDX_W)
        pltpu.sync_copy(idx_hbm.at[wid], idx_v)

        def fire(g, b):
            for j in range(K):
                pltpu.async_copy(
                    table_hbm.at[idx_v.at[g * K + j]],
                    rows_v.at[b, pl.ds(j * IDX_W, IDX_W)],
                    gsem.at[b],
                )

        def drain(b):
            for j in range(K):
                pltpu.make_async_copy(
                    table_hbm.at[idx_v.at[0]],
                    rows_v.at[b, pl.ds(j * IDX_W, IDX_W)],
                    gsem.at[b],
                ).wait()

        def scat_start(g, b):
            pltpu.async_copy(
                rows_v.at[b],
                out_hbm.at[pl.ds(base + g * grp, grp)],
                ssem.at[b],
            )

        def scat_wait(b):
            pltpu.make_async_copy(
                rows_v.at[b],
                out_hbm.at[pl.ds(base, grp)],
                ssem.at[b],
            ).wait()

        for b in range(NB - 1):
            fire(b, b)

        def outer(it, carry):
            for b in range(NB):
                g = it * NB + b
                bp = (b - 1) % NB
                drain(b)

                @pl.when(g >= 1)
                def _():
                    scat_wait(bp)

                @pl.when(g + NB - 1 <= ng - 1)
                def _():
                    fire(g + NB - 1, bp)

                scat_start(g, b)
            return carry

        lax.fori_loop(0, ng // NB, outer, 0)
        scat_wait((ng - 1) % NB)

    return lookup


def kernel(table, text_seqs):
    batch, seq = text_seqs.shape
    n = batch * seq
    assert n % (NW * IDX_W) == 0
    steps = n // (NW * IDX_W)
    assert steps % (K * NB) == 0
    idx = text_seqs.reshape(NW, steps, IDX_W)
    out = _make_lookup(steps)(table, idx)
    return out.reshape(batch, seq, DIM)


# speedup vs baseline: 14.4145x; 4.7801x over previous
import functools

import jax
import jax.numpy as jnp
from jax import lax
from jax.experimental import pallas as pl
from jax.experimental.pallas import tpu as pltpu
from jax.experimental.pallas import tpu_sc as plsc

DIM = 32
CB = 128
L = 16

_info = plsc.get_sparse_core_info()
NC = _info.num_cores
NS = _info.num_subcores
NW = NC * NS


def _make_lookup(vocab, batch, seq):
    mesh = plsc.VectorSubcoreMesh(core_axis_name="c", subcore_axis_name="s")
    nchunk = batch // CB

    @functools.partial(
        pl.kernel,
        mesh=mesh,
        compiler_params=pltpu.CompilerParams(needs_layout_passes=False),
        out_type=jax.ShapeDtypeStruct((seq, DIM, batch), jnp.float32),
        scratch_types=[
            pltpu.VMEM((vocab,), jnp.float32),
            pltpu.VMEM((seq, CB), jnp.int32),
            pltpu.VMEM((seq, CB), jnp.int32),
            pltpu.VMEM((seq, CB), jnp.float32),
            pltpu.VMEM((seq, CB), jnp.float32),
            pltpu.SemaphoreType.DMA((2,)),
            pltpu.SemaphoreType.DMA((2,)),
        ],
    )
    def lookup(
        tableT_hbm, seqsT_hbm, outp_hbm, trow_v, idx0, idx1, ob0, ob1, isem, osem
    ):
        d = lax.axis_index("s") * NC + lax.axis_index("c")
        pltpu.sync_copy(tableT_hbm.at[d], trow_v)
        bufs = ((idx0, ob0), (idx1, ob1))

        def fire_idx(g, b):
            pltpu.async_copy(
                seqsT_hbm.at[:, pl.ds(g * CB, CB)], bufs[b][0], isem.at[b]
            )

        def drain_idx(b):
            pltpu.make_async_copy(
                seqsT_hbm.at[:, pl.ds(0, CB)], bufs[b][0], isem.at[b]
            ).wait()

        def out_start(g, b):
            pltpu.async_copy(
                bufs[b][1], outp_hbm.at[:, d, pl.ds(g * CB, CB)], osem.at[b]
            )

        def out_wait(b):
            pltpu.make_async_copy(
                bufs[b][1], outp_hbm.at[:, d, pl.ds(0, CB)], osem.at[b]
            ).wait()

        fire_idx(0, 0)

        def outer(it, carry):
            for b in range(2):
                g = it * 2 + b
                idx_v, obuf_v = bufs[b]
                drain_idx(b)

                @pl.when(g + 1 <= nchunk - 1)
                def _():
                    fire_idx(g + 1, 1 - b)

                @pl.when(g >= 2)
                def _():
                    out_wait(b)

                def row(s, carry2):
                    for j in range(CB // L):
                        vidx = idx_v[s, pl.ds(j * L, L)]
                        vals = plsc.load_gather(trow_v, [vidx])
                        obuf_v[s, pl.ds(j * L, L)] = vals
                    return carry2

                lax.fori_loop(0, seq, row, 0)
                out_start(g, b)
            return carry

        lax.fori_loop(0, nchunk // 2, outer, 0)
        out_wait(0)
        out_wait(1)

    return lookup


def kernel(table, text_seqs):
    batch, seq = text_seqs.shape
    vocab = table.shape[0]
    assert batch % (2 * CB) == 0
    out_p = _make_lookup(vocab, batch, seq)(table.T, text_seqs.T)
    return jnp.transpose(out_p, (2, 0, 1))


# speedup vs baseline: 19.4583x; 1.3499x over previous
import functools

import jax
import jax.numpy as jnp
from jax import lax
from jax.experimental import pallas as pl
from jax.experimental.pallas import tpu as pltpu
from jax.experimental.pallas import tpu_sc as plsc

DIM = 32
CB = 128
L = 16

_info = plsc.get_sparse_core_info()
NC = _info.num_cores
NS = _info.num_subcores
NW = NC * NS


def _make_lookup(vocab, batch, seq):
    mesh = plsc.VectorSubcoreMesh(core_axis_name="c", subcore_axis_name="s")
    nchunk = batch // CB

    @functools.partial(
        pl.kernel,
        mesh=mesh,
        compiler_params=pltpu.CompilerParams(needs_layout_passes=False),
        out_type=jax.ShapeDtypeStruct((seq, DIM, batch), jnp.float32),
        scratch_types=[
            pltpu.VMEM((vocab,), jnp.float32),
            pltpu.VMEM((seq, CB), jnp.int32),
            pltpu.VMEM((seq, CB), jnp.int32),
            pltpu.VMEM((seq, CB), jnp.float32),
            pltpu.VMEM((seq, CB), jnp.float32),
            pltpu.SemaphoreType.DMA((2,)),
            pltpu.SemaphoreType.DMA((2,)),
        ],
    )
    def lookup(
        tableT_hbm, seqsT_hbm, outp_hbm, trow_v, idx0, idx1, ob0, ob1, isem, osem
    ):
        d = lax.axis_index("s") * NC + lax.axis_index("c")
        pltpu.sync_copy(tableT_hbm.at[d], trow_v)
        bufs = ((idx0, ob0), (idx1, ob1))

        def fire_idx(g, b):
            pltpu.async_copy(
                seqsT_hbm.at[:, pl.ds(g * CB, CB)], bufs[b][0], isem.at[b]
            )

        def drain_idx(b):
            pltpu.make_async_copy(
                seqsT_hbm.at[:, pl.ds(0, CB)], bufs[b][0], isem.at[b]
            ).wait()

        def out_start(g, b):
            pltpu.async_copy(
                bufs[b][1], outp_hbm.at[:, d, pl.ds(g * CB, CB)], osem.at[b]
            )

        def out_wait(b):
            pltpu.make_async_copy(
                bufs[b][1], outp_hbm.at[:, d, pl.ds(0, CB)], osem.at[b]
            ).wait()

        fire_idx(0, 0)

        def outer(it, carry):
            for b in range(2):
                g = it * 2 + b
                idx_v, obuf_v = bufs[b]
                drain_idx(b)

                @pl.when(g + 1 <= nchunk - 1)
                def _():
                    fire_idx(g + 1, 1 - b)

                @pl.when(g >= 2)
                def _():
                    out_wait(b)

                @plsc.parallel_loop(0, seq, unroll=2)
                def _row(s):
                    for j in range(CB // L):
                        vidx = idx_v[s, pl.ds(j * L, L)]
                        vals = plsc.load_gather(trow_v, [vidx])
                        obuf_v[s, pl.ds(j * L, L)] = vals
                out_start(g, b)
            return carry

        lax.fori_loop(0, nchunk // 2, outer, 0)
        out_wait(0)
        out_wait(1)

    return lookup


def kernel(table, text_seqs):
    batch, seq = text_seqs.shape
    vocab = table.shape[0]
    assert batch % (2 * CB) == 0
    out_p = _make_lookup(vocab, batch, seq)(table.T, text_seqs.T)
    return jnp.transpose(out_p, (2, 0, 1))
